# baseline (speedup 1.0000x reference)
import functools

import jax
import jax.numpy as jnp
from jax import lax
from jax.experimental import pallas as pl
from jax.experimental.pallas import tpu as pltpu
from jax.experimental.pallas import tpu_sc as plsc

_B, _T, _C = 8, 4608, 768
_L = 16
_NW = 32
_CB = 128
_NCG = _C // _CB
_SCT = 2880
_NU = 10
_CHT = _SCT // _NU
_TASKS = _B * _NCG * _NU
_TPW = _TASKS // _NW
_NSUB = _CB // _L
_TCT = _T - _SCT
_TCBLK = 864
_NTCB = _TCT // _TCBLK

_SORT_NET = [
    [(0, 13), (1, 12), (2, 15), (3, 14), (4, 8), (5, 6), (7, 11), (9, 10)],
    [(0, 5), (1, 7), (2, 9), (3, 4), (6, 13), (8, 14), (10, 15), (11, 12)],
    [(0, 1), (2, 3), (4, 5), (6, 8), (7, 9), (10, 11), (12, 13), (14, 15)],
    [(0, 2), (1, 3), (4, 10), (5, 11), (6, 7), (8, 9), (12, 14), (13, 15)],
    [(1, 2), (3, 12), (4, 6), (5, 7), (8, 10), (9, 11), (13, 14)],
    [(1, 4), (2, 6), (5, 8), (7, 10), (9, 13), (11, 14)],
    [(2, 4), (3, 6), (9, 12), (11, 13)],
    [(3, 5), (6, 8), (7, 9), (10, 12)],
    [(3, 4), (5, 6), (7, 8), (9, 10), (11, 12)],
    [(6, 7), (8, 9)],
]
_HALF_CLEAN = [[(i, i ^ j) for i in range(16) if (i ^ j) > i] for j in (8, 4, 2, 1)]


def _sort16_desc(regs):
    regs = list(regs)
    for st in _SORT_NET:
        for (i, l) in st:
            a, b = regs[i], regs[l]
            regs[i] = jnp.maximum(a, b)
            regs[l] = jnp.minimum(a, b)
    return regs


def _merge_topk(state, new_sorted):
    m = [jnp.maximum(new_sorted[i], state[15 - i]) for i in range(16)]
    for st in _HALF_CLEAN:
        for (i, l) in st:
            a, b = m[i], m[l]
            m[i], m[l] = jnp.maximum(a, b), jnp.minimum(a, b)
    return m


def _sc_topk_body(l4_hbm, out_hbm, buf0, buf1, res, sem0, sem1, *, nc):
    wid = lax.axis_index("s") * nc + lax.axis_index("c")
    bufs = (buf0, buf1)
    sems = (sem0, sem1)

    def src_slice(q):
        t = q * _NW + wid
        r = t % (_NCG * _NU)
        b = t // (_NCG * _NU)
        cg = r // _NU
        unit = r % _NU
        return l4_hbm.at[b, pl.ds(_TCT + unit * _CHT, _CHT), pl.ds(cg * _CB, _CB)]

    pltpu.make_async_copy(src_slice(0), buf0, sem0).start()
    neg = jnp.full((_L,), -jnp.inf, jnp.float32)

    def task_work(q, p):
        q = jnp.int32(q)
        buf = bufs[p]
        pltpu.make_async_copy(src_slice(q), buf, sems[p]).wait()

        @pl.when(q + 1 < _TPW)
        def _prefetch():
            pltpu.make_async_copy(src_slice(q + 1), bufs[1 - p], sems[1 - p]).start()

        for sub in range(_NSUB):
            c0 = sub * _L
            state = tuple(neg for _ in range(_L))

            def blk(j, st):
                regs = [buf[j * _L + k, pl.ds(c0, _L)] for k in range(_L)]
                return tuple(_merge_topk(list(st), _sort16_desc(regs)))

            state = lax.fori_loop(0, _CHT // _L, blk, state, unroll=2)
            for s in range(_L):
                res[q, s, pl.ds(c0, _L)] = state[s]

    def dstep_body(d, carry):
        task_work(d * 2, 0)
        task_work(d * 2 + 1, 1)
        return carry

    lax.fori_loop(0, _TPW // 2, dstep_body, 0)
    if _TPW % 2:
        task_work(_TPW - 1, 0)
    pltpu.sync_copy(res, out_hbm.at[:, wid, :, :])


def _sc_topk(l4f):
    mesh = plsc.VectorSubcoreMesh(core_axis_name="c", subcore_axis_name="s")
    kern = pl.kernel(
        functools.partial(_sc_topk_body, nc=2),
        out_type=jax.ShapeDtypeStruct((_TPW, _NW, _L, _CB), jnp.float32),
        mesh=mesh,
        scratch_types=[
            pltpu.VMEM((_CHT, _CB), jnp.float32),
            pltpu.VMEM((_CHT, _CB), jnp.float32),
            pltpu.VMEM((_TPW, _L, _CB), jnp.float32),
            pltpu.SemaphoreType.DMA,
            pltpu.SemaphoreType.DMA,
        ],
    )
    out = kern(l4f)
    return out.reshape(_TASKS, _L, _CB).reshape(_B, _NCG, _NU, _L, _CB)


def _layer_norm(x, g, b, eps=1e-5):
    m = x.mean(axis=-1, keepdims=True)
    v = ((x - m) ** 2).mean(axis=-1, keepdims=True)
    return (x - m) / jnp.sqrt(v + eps) * g + b


def _projector(x, W1, b1, g1, be1, W2, b2, g2, be2):
    h = jnp.dot(x, W1, preferred_element_type=jnp.float32,
                precision=lax.Precision.HIGHEST) + b1
    h = _layer_norm(h, g1, be1)
    h = 0.5 * h * (1.0 + lax.erf(h * (2.0 ** -0.5)))
    h = jnp.dot(h, W2, preferred_element_type=jnp.float32,
                precision=lax.Precision.HIGHEST) + b2
    h = _layer_norm(h, g2, be2)
    n = jnp.maximum(jnp.sqrt(jnp.sum(h * h, axis=-1, keepdims=True)), 1e-12)
    return h / n


_TB = 768
_NSTEPS = _T // _TB


def _mean_body(l12_ref, out_ref):
    i = pl.program_id(0)

    @pl.when(i == 0)
    def _init():
        out_ref[...] = jnp.zeros_like(out_ref)

    out_ref[...] += jnp.sum(l12_ref[...], axis=1)

    @pl.when(i == _NSTEPS - 1)
    def _fin():
        out_ref[...] *= (1.0 / _T)


def _tc_mean(l12f):
    return pl.pallas_call(
        _mean_body,
        grid=(_NSTEPS,),
        in_specs=[pl.BlockSpec((_B, _TB, _C), lambda i: (0, i, 0))],
        out_specs=pl.BlockSpec((_B, _C), lambda i: (0, 0)),
        out_shape=jax.ShapeDtypeStruct((_B, _C), jnp.float32),
    )(l12f)


def _tc_topk_body(x_ref, out_ref):
    x = x_ref[...]
    w = _TCBLK // _L
    slots = _sort16_desc([x[:, k * w:(k + 1) * w, :] for k in range(_L)])
    while w > 1:
        h = w // 2
        m = [jnp.maximum(slots[s][:, :h], slots[_L - 1 - s][:, h:2 * h])
             for s in range(_L)]
        for st in _HALF_CLEAN:
            for (i, l) in st:
                a, b = m[i], m[l]
                m[i], m[l] = jnp.maximum(a, b), jnp.minimum(a, b)
        if w % 2:
            slots = [jnp.concatenate([m[s], slots[s][:, 2 * h:]], axis=1)
                     for s in range(_L)]
            w = h + 1
        else:
            slots = m
            w = h
    for s in range(_L):
        out_ref[:, 0, s, :] = slots[s][:, 0, :]


def _tc_topk(l4f):
    return pl.pallas_call(
        _tc_topk_body,
        grid=(_NTCB,),
        in_specs=[pl.BlockSpec((_B, _TCBLK, _C), lambda j: (0, j, 0))],
        out_specs=pl.BlockSpec((_B, 1, _L, _C), lambda j: (0, j, 0, 0)),
        out_shape=jax.ShapeDtypeStruct((_B, _NTCB, _L, _C), jnp.float32),
    )(l4f)


def _merge_lists(A, B, clean=True):
    m = [jnp.maximum(A[s], B[_L - 1 - s]) for s in range(_L)]
    if clean:
        for st in _HALF_CLEAN:
            for (i, l) in st:
                a, b = m[i], m[l]
                m[i], m[l] = jnp.maximum(a, b), jnp.minimum(a, b)
    return m


def _mlp_body(sc_ref, tc_ref, zsraw,
              tW1, tb1, tg1, tbe1, tW2, tb2, tg2, tbe2,
              sW1, sb1, sg1, sbe1, sW2, sb2, sg2, sbe2,
              zt_out, zs_out):
    cols = []
    for cg in range(_NCG):
        lists = [[sc_ref[:, cg, u, s, :] for s in range(_L)]
                 for u in range(_NU)]
        lists += [[tc_ref[:, j, s, pl.ds(cg * _CB, _CB)] for s in range(_L)]
                  for j in range(_NTCB)]
        while len(lists) > 2:
            nxt = [_merge_lists(lists[a], lists[a + 1])
                   for a in range(0, len(lists) - 1, 2)]
            if len(lists) % 2:
                nxt.append(lists[-1])
            lists = nxt
        final = _merge_lists(lists[0], lists[1], clean=False)
        acc = final[0]
        for s in range(1, _L):
            acc = acc + final[s]
        cols.append(acc * (1.0 / _L))
    ztraw = jnp.concatenate(cols, axis=-1)
    zt_out[...] = _projector(ztraw, tW1[...], tb1[...], tg1[...],
                             tbe1[...], tW2[...], tb2[...], tg2[...], tbe2[...])
    zs_out[...] = _projector(zsraw[...], sW1[...], sb1[...], sg1[...],
                             sbe1[...], sW2[...], sb2[...], sg2[...], sbe2[...])


def _tc_mlp(sc_lists, tc_lists, zstr_raw, tW1, tb1, tg1, tbe1, tW2, tb2, tg2,
            tbe2, sW1, sb1, sg1, sbe1, sW2, sb2, sg2, sbe2):
    d2 = tW2.shape[1]
    return pl.pallas_call(
        _mlp_body,
        out_shape=[
            jax.ShapeDtypeStruct((_B, d2), jnp.float32),
            jax.ShapeDtypeStruct((_B, d2), jnp.float32),
        ],
    )(sc_lists, tc_lists, zstr_raw, tW1, tb1, tg1, tbe1, tW2, tb2, tg2, tbe2,
      sW1, sb1, sg1, sbe1, sW2, sb2, sg2, sbe2)


def kernel(l4, l12, tW1, tb1, tg1, tbeta1, tW2, tb2, tg2, tbeta2,
           sW1, sb1, sg1, sbeta1, sW2, sb2, sg2, sbeta2):
    B, N, S, C = l4.shape
    l4f = l4.reshape(B, N * S, C)
    l12f = l12.reshape(B, N * S, C)
    sc_lists = _sc_topk(l4f)
    tc_lists = _tc_topk(l4f)
    zstr_raw = _tc_mean(l12f)
    zt, zs = _tc_mlp(sc_lists, tc_lists, zstr_raw, tW1, tb1, tg1, tbeta1,
                     tW2, tb2, tg2, tbeta2, sW1, sb1, sg1, sbeta1, sW2, sb2,
                     sg2, sbeta2)
    return (zt, zs)


# speedup vs baseline: 1.0271x; 1.0271x over previous
import functools

import jax
import jax.numpy as jnp
from jax import lax
from jax.experimental import pallas as pl
from jax.experimental.pallas import tpu as pltpu
from jax.experimental.pallas import tpu_sc as plsc

_B, _T, _C = 8, 4608, 768
_L = 16
_NW = 32
_CB = 128
_NCG = _C // _CB
_SCT = 2560
_NU = 10
_CHT = _SCT // _NU
_TASKS = _B * _NCG * _NU
_TPW = _TASKS // _NW
_NSUB = _CB // _L
_TCT = _T - _SCT
_TCBLK = 512
_NTCB = _TCT // _TCBLK

_SORT_NET = [
    [(0, 13), (1, 12), (2, 15), (3, 14), (4, 8), (5, 6), (7, 11), (9, 10)],
    [(0, 5), (1, 7), (2, 9), (3, 4), (6, 13), (8, 14), (10, 15), (11, 12)],
    [(0, 1), (2, 3), (4, 5), (6, 8), (7, 9), (10, 11), (12, 13), (14, 15)],
    [(0, 2), (1, 3), (4, 10), (5, 11), (6, 7), (8, 9), (12, 14), (13, 15)],
    [(1, 2), (3, 12), (4, 6), (5, 7), (8, 10), (9, 11), (13, 14)],
    [(1, 4), (2, 6), (5, 8), (7, 10), (9, 13), (11, 14)],
    [(2, 4), (3, 6), (9, 12), (11, 13)],
    [(3, 5), (6, 8), (7, 9), (10, 12)],
    [(3, 4), (5, 6), (7, 8), (9, 10), (11, 12)],
    [(6, 7), (8, 9)],
]
_HALF_CLEAN = [[(i, i ^ j) for i in range(16) if (i ^ j) > i] for j in (8, 4, 2, 1)]


def _sort16_desc(regs):
    regs = list(regs)
    for st in _SORT_NET:
        for (i, l) in st:
            a, b = regs[i], regs[l]
            regs[i] = jnp.maximum(a, b)
            regs[l] = jnp.minimum(a, b)
    return regs


def _merge_topk(state, new_sorted):
    m = [jnp.maximum(new_sorted[i], state[15 - i]) for i in range(16)]
    for st in _HALF_CLEAN:
        for (i, l) in st:
            a, b = m[i], m[l]
            m[i], m[l] = jnp.maximum(a, b), jnp.minimum(a, b)
    return m


def _sc_topk_body(l4_hbm, out_hbm, buf0, buf1, res, sem0, sem1, *, nc):
    wid = lax.axis_index("s") * nc + lax.axis_index("c")
    bufs = (buf0, buf1)
    sems = (sem0, sem1)

    def src_slice(q):
        t = q * _NW + wid
        r = t % (_NCG * _NU)
        b = t // (_NCG * _NU)
        cg = r // _NU
        unit = r % _NU
        return l4_hbm.at[b, pl.ds(_TCT + unit * _CHT, _CHT), pl.ds(cg * _CB, _CB)]

    pltpu.make_async_copy(src_slice(0), buf0, sem0).start()
    neg = jnp.full((_L,), -jnp.inf, jnp.float32)

    def task_work(q, p):
        q = jnp.int32(q)
        buf = bufs[p]
        pltpu.make_async_copy(src_slice(q), buf, sems[p]).wait()

        @pl.when(q + 1 < _TPW)
        def _prefetch():
            pltpu.make_async_copy(src_slice(q + 1), bufs[1 - p], sems[1 - p]).start()

        for sub in range(_NSUB):
            c0 = sub * _L
            state = tuple(neg for _ in range(_L))

            def blk(j, st):
                regs = [buf[j * _L + k, pl.ds(c0, _L)] for k in range(_L)]
                return tuple(_merge_topk(list(st), _sort16_desc(regs)))

            state = lax.fori_loop(0, _CHT // _L, blk, state, unroll=2)
            for s in range(_L):
                res[q, s, pl.ds(c0, _L)] = state[s]

    def dstep_body(d, carry):
        task_work(d * 2, 0)
        task_work(d * 2 + 1, 1)
        return carry

    lax.fori_loop(0, _TPW // 2, dstep_body, 0)
    if _TPW % 2:
        task_work(_TPW - 1, 0)
    pltpu.sync_copy(res, out_hbm.at[:, wid, :, :])


def _sc_topk(l4f):
    mesh = plsc.VectorSubcoreMesh(core_axis_name="c", subcore_axis_name="s")
    kern = pl.kernel(
        functools.partial(_sc_topk_body, nc=2),
        out_type=jax.ShapeDtypeStruct((_TPW, _NW, _L, _CB), jnp.float32),
        mesh=mesh,
        scratch_types=[
            pltpu.VMEM((_CHT, _CB), jnp.float32),
            pltpu.VMEM((_CHT, _CB), jnp.float32),
            pltpu.VMEM((_TPW, _L, _CB), jnp.float32),
            pltpu.SemaphoreType.DMA,
            pltpu.SemaphoreType.DMA,
        ],
    )
    out = kern(l4f)
    return out.reshape(_TASKS, _L, _CB).reshape(_B, _NCG, _NU, _L, _CB)


def _layer_norm(x, g, b, eps=1e-5):
    m = x.mean(axis=-1, keepdims=True)
    v = ((x - m) ** 2).mean(axis=-1, keepdims=True)
    return (x - m) / jnp.sqrt(v + eps) * g + b


def _projector(x, W1, b1, g1, be1, W2, b2, g2, be2):
    h = jnp.dot(x, W1, preferred_element_type=jnp.float32,
                precision=lax.Precision.HIGHEST) + b1
    h = _layer_norm(h, g1, be1)
    h = 0.5 * h * (1.0 + lax.erf(h * (2.0 ** -0.5)))
    h = jnp.dot(h, W2, preferred_element_type=jnp.float32,
                precision=lax.Precision.HIGHEST) + b2
    h = _layer_norm(h, g2, be2)
    n = jnp.maximum(jnp.sqrt(jnp.sum(h * h, axis=-1, keepdims=True)), 1e-12)
    return h / n


_TB = 512
_NSTEPS = _T // _TB


def _mean_body(l12_ref, out_ref):
    i = pl.program_id(0)

    @pl.when(i == 0)
    def _init():
        out_ref[...] = jnp.zeros_like(out_ref)

    out_ref[...] += jnp.sum(l12_ref[...], axis=1)

    @pl.when(i == _NSTEPS - 1)
    def _fin():
        out_ref[...] *= (1.0 / _T)


def _tc_mean(l12f):
    return pl.pallas_call(
        _mean_body,
        grid=(_NSTEPS,),
        in_specs=[pl.BlockSpec((_B, _TB, _C), lambda i: (0, i, 0))],
        out_specs=pl.BlockSpec((_B, _C), lambda i: (0, 0)),
        out_shape=jax.ShapeDtypeStruct((_B, _C), jnp.float32),
    )(l12f)


def _tc_topk_body(x_ref, out_ref):
    x = x_ref[...]
    w = _TCBLK // _L
    slots = _sort16_desc([x[:, k * w:(k + 1) * w, :] for k in range(_L)])
    while w > 1:
        h = w // 2
        m = [jnp.maximum(slots[s][:, :h], slots[_L - 1 - s][:, h:2 * h])
             for s in range(_L)]
        for st in _HALF_CLEAN:
            for (i, l) in st:
                a, b = m[i], m[l]
                m[i], m[l] = jnp.maximum(a, b), jnp.minimum(a, b)
        if w % 2:
            slots = [jnp.concatenate([m[s], slots[s][:, 2 * h:]], axis=1)
                     for s in range(_L)]
            w = h + 1
        else:
            slots = m
            w = h
    for s in range(_L):
        out_ref[:, 0, s, :] = slots[s][:, 0, :]


def _tc_topk(l4f):
    return pl.pallas_call(
        _tc_topk_body,
        grid=(_NTCB,),
        in_specs=[pl.BlockSpec((_B, _TCBLK, _C), lambda j: (0, j, 0))],
        out_specs=pl.BlockSpec((_B, 1, _L, _C), lambda j: (0, j, 0, 0)),
        out_shape=jax.ShapeDtypeStruct((_B, _NTCB, _L, _C), jnp.float32),
    )(l4f)


def _merge_lists(A, B, clean=True):
    m = [jnp.maximum(A[s], B[_L - 1 - s]) for s in range(_L)]
    if clean:
        for st in _HALF_CLEAN:
            for (i, l) in st:
                a, b = m[i], m[l]
                m[i], m[l] = jnp.maximum(a, b), jnp.minimum(a, b)
    return m


def _mlp_body(sc_ref, tc_ref, zsraw,
              tW1, tb1, tg1, tbe1, tW2, tb2, tg2, tbe2,
              sW1, sb1, sg1, sbe1, sW2, sb2, sg2, sbe2,
              zt_out, zs_out):
    cols = []
    for cg in range(_NCG):
        lists = [[sc_ref[:, cg, u, s, :] for s in range(_L)]
                 for u in range(_NU)]
        lists += [[tc_ref[:, j, s, pl.ds(cg * _CB, _CB)] for s in range(_L)]
                  for j in range(_NTCB)]
        while len(lists) > 2:
            nxt = [_merge_lists(lists[a], lists[a + 1])
                   for a in range(0, len(lists) - 1, 2)]
            if len(lists) % 2:
                nxt.append(lists[-1])
            lists = nxt
        final = _merge_lists(lists[0], lists[1], clean=False)
        acc = final[0]
        for s in range(1, _L):
            acc = acc + final[s]
        cols.append(acc * (1.0 / _L))
    ztraw = jnp.concatenate(cols, axis=-1)
    zt_out[...] = _projector(ztraw, tW1[...], tb1[...], tg1[...],
                             tbe1[...], tW2[...], tb2[...], tg2[...], tbe2[...])
    zs_out[...] = _projector(zsraw[...], sW1[...], sb1[...], sg1[...],
                             sbe1[...], sW2[...], sb2[...], sg2[...], sbe2[...])


def _tc_mlp(sc_lists, tc_lists, zstr_raw, tW1, tb1, tg1, tbe1, tW2, tb2, tg2,
            tbe2, sW1, sb1, sg1, sbe1, sW2, sb2, sg2, sbe2):
    d2 = tW2.shape[1]
    return pl.pallas_call(
        _mlp_body,
        out_shape=[
            jax.ShapeDtypeStruct((_B, d2), jnp.float32),
            jax.ShapeDtypeStruct((_B, d2), jnp.float32),
        ],
    )(sc_lists, tc_lists, zstr_raw, tW1, tb1, tg1, tbe1, tW2, tb2, tg2, tbe2,
      sW1, sb1, sg1, sbe1, sW2, sb2, sg2, sbe2)


def kernel(l4, l12, tW1, tb1, tg1, tbeta1, tW2, tb2, tg2, tbeta2,
           sW1, sb1, sg1, sbeta1, sW2, sb2, sg2, sbeta2):
    B, N, S, C = l4.shape
    l4f = l4.reshape(B, N * S, C)
    l12f = l12.reshape(B, N * S, C)
    sc_lists = _sc_topk(l4f)
    tc_lists = _tc_topk(l4f)
    zstr_raw = _tc_mean(l12f)
    zt, zs = _tc_mlp(sc_lists, tc_lists, zstr_raw, tW1, tb1, tg1, tbeta1,
                     tW2, tb2, tg2, tbeta2, sW1, sb1, sg1, sbeta1, sW2, sb2,
                     sg2, sbeta2)
    return (zt, zs)


# speedup vs baseline: 1.0915x; 1.0627x over previous
import functools

import jax
import jax.numpy as jnp
from jax import lax
from jax.experimental import pallas as pl
from jax.experimental.pallas import tpu as pltpu
from jax.experimental.pallas import tpu_sc as plsc

_B, _T, _C = 8, 4608, 768
_L = 16
_NW = 32
_CB = 128
_NCG = _C // _CB
_SCT = 2048
_NU = 8
_CHT = _SCT // _NU
_TASKS = _B * _NCG * _NU
_TPW = _TASKS // _NW
_NSUB = _CB // _L
_TCT = _T - _SCT
_TCBLK = 512
_NTCB = _TCT // _TCBLK

_SORT_NET = [
    [(0, 13), (1, 12), (2, 15), (3, 14), (4, 8), (5, 6), (7, 11), (9, 10)],
    [(0, 5), (1, 7), (2, 9), (3, 4), (6, 13), (8, 14), (10, 15), (11, 12)],
    [(0, 1), (2, 3), (4, 5), (6, 8), (7, 9), (10, 11), (12, 13), (14, 15)],
    [(0, 2), (1, 3), (4, 10), (5, 11), (6, 7), (8, 9), (12, 14), (13, 15)],
    [(1, 2), (3, 12), (4, 6), (5, 7), (8, 10), (9, 11), (13, 14)],
    [(1, 4), (2, 6), (5, 8), (7, 10), (9, 13), (11, 14)],
    [(2, 4), (3, 6), (9, 12), (11, 13)],
    [(3, 5), (6, 8), (7, 9), (10, 12)],
    [(3, 4), (5, 6), (7, 8), (9, 10), (11, 12)],
    [(6, 7), (8, 9)],
]
_HALF_CLEAN = [[(i, i ^ j) for i in range(16) if (i ^ j) > i] for j in (8, 4, 2, 1)]


def _sort16_desc(regs):
    regs = list(regs)
    for st in _SORT_NET:
        for (i, l) in st:
            a, b = regs[i], regs[l]
            regs[i] = jnp.maximum(a, b)
            regs[l] = jnp.minimum(a, b)
    return regs


def _merge_topk(state, new_sorted):
    m = [jnp.maximum(new_sorted[i], state[15 - i]) for i in range(16)]
    for st in _HALF_CLEAN:
        for (i, l) in st:
            a, b = m[i], m[l]
            m[i], m[l] = jnp.maximum(a, b), jnp.minimum(a, b)
    return m


def _sc_topk_body(l4_hbm, out_hbm, buf0, buf1, res, sem0, sem1, *, nc):
    wid = lax.axis_index("s") * nc + lax.axis_index("c")
    bufs = (buf0, buf1)
    sems = (sem0, sem1)

    def src_slice(q):
        t = q * _NW + wid
        r = t % (_NCG * _NU)
        b = t // (_NCG * _NU)
        cg = r // _NU
        unit = r % _NU
        return l4_hbm.at[b, pl.ds(_TCT + unit * _CHT, _CHT), pl.ds(cg * _CB, _CB)]

    pltpu.make_async_copy(src_slice(0), buf0, sem0).start()
    neg = jnp.full((_L,), -jnp.inf, jnp.float32)

    def task_work(q, p):
        q = jnp.int32(q)
        buf = bufs[p]
        pltpu.make_async_copy(src_slice(q), buf, sems[p]).wait()

        @pl.when(q + 1 < _TPW)
        def _prefetch():
            pltpu.make_async_copy(src_slice(q + 1), bufs[1 - p], sems[1 - p]).start()

        for sub in range(_NSUB):
            c0 = sub * _L
            state = tuple(neg for _ in range(_L))

            def blk(j, st):
                regs = [buf[j * _L + k, pl.ds(c0, _L)] for k in range(_L)]
                return tuple(_merge_topk(list(st), _sort16_desc(regs)))

            state = lax.fori_loop(0, _CHT // _L, blk, state, unroll=2)
            for s in range(_L):
                res[q, s, pl.ds(c0, _L)] = state[s]

    def dstep_body(d, carry):
        task_work(d * 2, 0)
        task_work(d * 2 + 1, 1)
        return carry

    lax.fori_loop(0, _TPW // 2, dstep_body, 0)
    if _TPW % 2:
        task_work(_TPW - 1, 0)
    pltpu.sync_copy(res, out_hbm.at[:, wid, :, :])


def _sc_topk(l4f):
    mesh = plsc.VectorSubcoreMesh(core_axis_name="c", subcore_axis_name="s")
    kern = pl.kernel(
        functools.partial(_sc_topk_body, nc=2),
        out_type=jax.ShapeDtypeStruct((_TPW, _NW, _L, _CB), jnp.float32),
        mesh=mesh,
        scratch_types=[
            pltpu.VMEM((_CHT, _CB), jnp.float32),
            pltpu.VMEM((_CHT, _CB), jnp.float32),
            pltpu.VMEM((_TPW, _L, _CB), jnp.float32),
            pltpu.SemaphoreType.DMA,
            pltpu.SemaphoreType.DMA,
        ],
    )
    out = kern(l4f)
    return out.reshape(_TASKS, _L, _CB).reshape(_B, _NCG, _NU, _L, _CB)


def _layer_norm(x, g, b, eps=1e-5):
    m = x.mean(axis=-1, keepdims=True)
    v = ((x - m) ** 2).mean(axis=-1, keepdims=True)
    return (x - m) / jnp.sqrt(v + eps) * g + b


def _projector(x, W1, b1, g1, be1, W2, b2, g2, be2):
    h = jnp.dot(x, W1, preferred_element_type=jnp.float32,
                precision=lax.Precision.HIGHEST) + b1
    h = _layer_norm(h, g1, be1)
    h = 0.5 * h * (1.0 + lax.erf(h * (2.0 ** -0.5)))
    h = jnp.dot(h, W2, preferred_element_type=jnp.float32,
                precision=lax.Precision.HIGHEST) + b2
    h = _layer_norm(h, g2, be2)
    n = jnp.maximum(jnp.sqrt(jnp.sum(h * h, axis=-1, keepdims=True)), 1e-12)
    return h / n


_TB = 512
_NSTEPS = _T // _TB


def _mean_body(l12_ref, out_ref):
    i = pl.program_id(0)

    @pl.when(i == 0)
    def _init():
        out_ref[...] = jnp.zeros_like(out_ref)

    out_ref[...] += jnp.sum(l12_ref[...], axis=1)

    @pl.when(i == _NSTEPS - 1)
    def _fin():
        out_ref[...] *= (1.0 / _T)


def _tc_mean(l12f):
    return pl.pallas_call(
        _mean_body,
        grid=(_NSTEPS,),
        in_specs=[pl.BlockSpec((_B, _TB, _C), lambda i: (0, i, 0))],
        out_specs=pl.BlockSpec((_B, _C), lambda i: (0, 0)),
        out_shape=jax.ShapeDtypeStruct((_B, _C), jnp.float32),
    )(l12f)


def _tc_topk_body(x_ref, out_ref):
    x = x_ref[...]
    w = _TCBLK // _L
    slots = _sort16_desc([x[:, k * w:(k + 1) * w, :] for k in range(_L)])
    while w > 1:
        h = w // 2
        m = [jnp.maximum(slots[s][:, :h], slots[_L - 1 - s][:, h:2 * h])
             for s in range(_L)]
        for st in _HALF_CLEAN:
            for (i, l) in st:
                a, b = m[i], m[l]
                m[i], m[l] = jnp.maximum(a, b), jnp.minimum(a, b)
        if w % 2:
            slots = [jnp.concatenate([m[s], slots[s][:, 2 * h:]], axis=1)
                     for s in range(_L)]
            w = h + 1
        else:
            slots = m
            w = h
    for s in range(_L):
        out_ref[:, 0, s, :] = slots[s][:, 0, :]


def _tc_topk(l4f):
    return pl.pallas_call(
        _tc_topk_body,
        grid=(_NTCB,),
        in_specs=[pl.BlockSpec((_B, _TCBLK, _C), lambda j: (0, j, 0))],
        out_specs=pl.BlockSpec((_B, 1, _L, _C), lambda j: (0, j, 0, 0)),
        out_shape=jax.ShapeDtypeStruct((_B, _NTCB, _L, _C), jnp.float32),
    )(l4f)


def _merge_lists(A, B, clean=True):
    m = [jnp.maximum(A[s], B[_L - 1 - s]) for s in range(_L)]
    if clean:
        for st in _HALF_CLEAN:
            for (i, l) in st:
                a, b = m[i], m[l]
                m[i], m[l] = jnp.maximum(a, b), jnp.minimum(a, b)
    return m


def _mlp_body(sc_ref, tc_ref, zsraw,
              tW1, tb1, tg1, tbe1, tW2, tb2, tg2, tbe2,
              sW1, sb1, sg1, sbe1, sW2, sb2, sg2, sbe2,
              zt_out, zs_out):
    cols = []
    for cg in range(_NCG):
        lists = [[sc_ref[:, cg, u, s, :] for s in range(_L)]
                 for u in range(_NU)]
        lists += [[tc_ref[:, j, s, pl.ds(cg * _CB, _CB)] for s in range(_L)]
                  for j in range(_NTCB)]
        while len(lists) > 2:
            nxt = [_merge_lists(lists[a], lists[a + 1])
                   for a in range(0, len(lists) - 1, 2)]
            if len(lists) % 2:
                nxt.append(lists[-1])
            lists = nxt
        final = _merge_lists(lists[0], lists[1], clean=False)
        acc = final[0]
        for s in range(1, _L):
            acc = acc + final[s]
        cols.append(acc * (1.0 / _L))
    ztraw = jnp.concatenate(cols, axis=-1)
    zt_out[...] = _projector(ztraw, tW1[...], tb1[...], tg1[...],
                             tbe1[...], tW2[...], tb2[...], tg2[...], tbe2[...])
    zs_out[...] = _projector(zsraw[...], sW1[...], sb1[...], sg1[...],
                             sbe1[...], sW2[...], sb2[...], sg2[...], sbe2[...])


def _tc_mlp(sc_lists, tc_lists, zstr_raw, tW1, tb1, tg1, tbe1, tW2, tb2, tg2,
            tbe2, sW1, sb1, sg1, sbe1, sW2, sb2, sg2, sbe2):
    d2 = tW2.shape[1]
    return pl.pallas_call(
        _mlp_body,
        out_shape=[
            jax.ShapeDtypeStruct((_B, d2), jnp.float32),
            jax.ShapeDtypeStruct((_B, d2), jnp.float32),
        ],
    )(sc_lists, tc_lists, zstr_raw, tW1, tb1, tg1, tbe1, tW2, tb2, tg2, tbe2,
      sW1, sb1, sg1, sbe1, sW2, sb2, sg2, sbe2)


def kernel(l4, l12, tW1, tb1, tg1, tbeta1, tW2, tb2, tg2, tbeta2,
           sW1, sb1, sg1, sbeta1, sW2, sb2, sg2, sbeta2):
    B, N, S, C = l4.shape
    l4f = l4.reshape(B, N * S, C)
    l12f = l12.reshape(B, N * S, C)
    sc_lists = _sc_topk(l4f)
    tc_lists = _tc_topk(l4f)
    zstr_raw = _tc_mean(l12f)
    zt, zs = _tc_mlp(sc_lists, tc_lists, zstr_raw, tW1, tb1, tg1, tbeta1,
                     tW2, tb2, tg2, tbeta2, sW1, sb1, sg1, sbeta1, sW2, sb2,
                     sg2, sbeta2)
    return (zt, zs)
